# baseline (speedup 1.0000x reference)
import functools

import jax
import jax.numpy as jnp
from jax import lax
from jax.experimental import pallas as pl
from jax.experimental.pallas import tpu as pltpu
from jax.experimental.pallas import tpu_sc as plsc

D3 = 128
BATCH = 16384
NC = 2
NS = 16
NW = NC * NS
BPW = BATCH // NW
ROWS = 3 * BPW
CW = 384
NCH = ROWS // CW
NBUF = 2
L = 16

_mesh = plsc.VectorSubcoreMesh(core_axis_name="c", subcore_axis_name="s")


@functools.partial(
    pl.kernel,
    mesh=_mesh,
    compiler_params=pltpu.CompilerParams(needs_layout_passes=False),
    out_type=jax.ShapeDtypeStruct((3 * BATCH, D3), jnp.float32),
    scratch_types=[
        pltpu.VMEM((BPW,), jnp.int32),
        pltpu.VMEM((BPW,), jnp.int32),
        pltpu.VMEM((BPW,), jnp.int32),
        pltpu.VMEM((ROWS,), jnp.int32),
        pltpu.VMEM((NBUF, CW, D3), jnp.float32),
        pltpu.SemaphoreType.DMA,
        pltpu.SemaphoreType.DMA,
    ],
)
def _pe3d(xh, yh, zh, th, out, xi, yi, zi, comb, rows, gsem, wsem):
    wid = lax.axis_index("s") * NC + lax.axis_index("c")
    base = wid * BPW
    pltpu.sync_copy(xh.at[pl.ds(base, BPW)], xi)
    pltpu.sync_copy(yh.at[pl.ds(base, BPW)], yi)
    pltpu.sync_copy(zh.at[pl.ds(base, BPW)], zi)
    lane3 = lax.iota(jnp.int32, L) * 3
    for g in range(BPW // L):
        sl = pl.ds(g * L, L)
        pos = lane3 + (3 * g * L)
        plsc.store_scatter(comb, [pos], xi[sl])
        plsc.store_scatter(comb, [pos + 1], yi[sl] + 256)
        plsc.store_scatter(comb, [pos + 2], zi[sl] + 512)
    obase = wid * ROWS
    writes = [None] * NCH
    for ci in range(NCH):
        b = ci % NBUF
        if ci >= NBUF:
            writes[ci - NBUF].wait()
        g = pltpu.async_copy(th.at[comb.at[pl.ds(ci * CW, CW)]], rows.at[b], gsem)
        g.wait()
        writes[ci] = pltpu.async_copy(
            rows.at[b], out.at[pl.ds(obase + ci * CW, CW)], wsem
        )
    for ci in range(NCH - NBUF, NCH):
        writes[ci].wait()


def kernel(x, y, z, x_pos, y_pos, z_pos):
    table = jnp.concatenate((x_pos, y_pos, z_pos), axis=0)
    out = _pe3d(
        x.astype(jnp.int32),
        y.astype(jnp.int32),
        z.astype(jnp.int32),
        table,
    )
    return out.reshape(BATCH, 3 * D3)


# speedup vs baseline: 1.5468x; 1.5468x over previous
import functools

import jax
import jax.numpy as jnp
from jax import lax
from jax.experimental import pallas as pl
from jax.experimental.pallas import tpu as pltpu
from jax.experimental.pallas import tpu_sc as plsc

D3 = 128
BATCH = 16384
NC = 2
NS = 16
NW = NC * NS
BPW = BATCH // NW
CH = 128
NCH = BPW // CH
NBUF = 2

_mesh = plsc.VectorSubcoreMesh(core_axis_name="c", subcore_axis_name="s")


@functools.partial(
    pl.kernel,
    mesh=_mesh,
    out_type=jax.ShapeDtypeStruct((BATCH, 3 * D3), jnp.float32),
    scratch_types=[
        pltpu.VMEM((BPW,), jnp.int32),
        pltpu.VMEM((BPW,), jnp.int32),
        pltpu.VMEM((BPW,), jnp.int32),
        pltpu.VMEM((NBUF, CH, D3), jnp.float32),
        pltpu.VMEM((NBUF, CH, D3), jnp.float32),
        pltpu.VMEM((NBUF, CH, D3), jnp.float32),
        pltpu.SemaphoreType.DMA,
        pltpu.SemaphoreType.DMA,
        pltpu.SemaphoreType.DMA,
        pltpu.SemaphoreType.DMA,
    ],
)
def _pe3d(xh, yh, zh, xt, yt, zt, out, xi, yi, zi, rx, ry, rz, g0, g1, w0, w1):
    gsems = (g0, g1)
    wsems = (w0, w1)
    wid = lax.axis_index("s") * NC + lax.axis_index("c")
    base = wid * BPW
    pltpu.sync_copy(xh.at[pl.ds(base, BPW)], xi)
    pltpu.sync_copy(yh.at[pl.ds(base, BPW)], yi)
    pltpu.sync_copy(zh.at[pl.ds(base, BPW)], zi)
    gathers = [None] * NCH
    writes = [None] * NCH

    def issue_writes(ci):
        for g in gathers[ci]:
            g.wait()
        b = ci % NBUF
        r0 = base + ci * CH
        ws = wsems[b]
        writes[ci] = (
            pltpu.async_copy(rx.at[b], out.at[pl.ds(r0, CH), pl.ds(0, D3)], ws),
            pltpu.async_copy(ry.at[b], out.at[pl.ds(r0, CH), pl.ds(D3, D3)], ws),
            pltpu.async_copy(rz.at[b], out.at[pl.ds(r0, CH), pl.ds(2 * D3, D3)], ws),
        )

    for ci in range(NCH):
        b = ci % NBUF
        if ci >= NBUF:
            for w in writes[ci - NBUF]:
                w.wait()
        sl = pl.ds(ci * CH, CH)
        gs = gsems[b]
        gathers[ci] = (
            pltpu.async_copy(xt.at[xi.at[sl]], rx.at[b], gs),
            pltpu.async_copy(yt.at[yi.at[sl]], ry.at[b], gs),
            pltpu.async_copy(zt.at[zi.at[sl]], rz.at[b], gs),
        )
        if ci >= 1:
            issue_writes(ci - 1)
    issue_writes(NCH - 1)
    for ci in range(NCH - NBUF, NCH):
        for w in writes[ci]:
            w.wait()


def kernel(x, y, z, x_pos, y_pos, z_pos):
    return _pe3d(
        x.astype(jnp.int32),
        y.astype(jnp.int32),
        z.astype(jnp.int32),
        x_pos,
        y_pos,
        z_pos,
    )
